# baseline (speedup 1.0000x reference)
import functools

import jax
import jax.numpy as jnp
from jax import lax
from jax.experimental import pallas as pl
from jax.experimental.pallas import tpu as pltpu
from jax.experimental.pallas import tpu_sc as plsc

_VOCAB = 1000000
_EMBED = 32
_BATCH = 16384
_HIST = 50

_NC = 2
_NS = 16
_NW = _NC * _NS

_G = 4
_CHUNK = _G * 128
_NBTG = _BATCH // _CHUNK
_NUNIT = _HIST * _NBTG
_U_PER_W = _NUNIT // _NW

_mesh = plsc.VectorSubcoreMesh(core_axis_name="c", subcore_axis_name="s")


@functools.partial(
    pl.kernel,
    mesh=_mesh,
    out_type=jax.ShapeDtypeStruct(
        (_HIST, _EMBED // 8, _BATCH // 128, 8, 128), jnp.float32
    ),
    scratch_types=[
        pltpu.VMEM((_CHUNK,), jnp.int32),
        pltpu.VMEM((_CHUNK,), jnp.int32),
        pltpu.VMEM((_CHUNK, _EMBED), jnp.float32),
        pltpu.VMEM((_CHUNK, _EMBED), jnp.float32),
        pltpu.VMEM((_G, _EMBED, 129), jnp.float32),
        pltpu.VMEM((_G, _EMBED, 129), jnp.float32),
        pltpu.SemaphoreType.DMA,
        pltpu.SemaphoreType.DMA,
        pltpu.SemaphoreType.DMA,
        pltpu.SemaphoreType.DMA,
    ],
    compiler_params=pltpu.CompilerParams(
        use_tc_tiling_on_sc=False, needs_layout_passes=False
    ),
)
def _gather_kernel(idx_hbm, table_hbm, out_hbm, i0, i1, r0, r1, t0, t1,
                   g0, g1, os0, os1):
    wid = lax.axis_index("s") * _NC + lax.axis_index("c")
    u0 = wid * _U_PER_W
    lanes = lax.iota(jnp.int32, 16)

    def start_gather(k, ibuf, rbuf, gsem):
        u = u0 + k
        h = u // _NBTG
        btg = u % _NBTG
        pltpu.sync_copy(idx_hbm.at[h].at[pl.ds(btg * _CHUNK, _CHUNK)], ibuf)
        return pltpu.async_copy(table_hbm.at[ibuf], rbuf, gsem)

    def wait_gather(rbuf, gsem):
        pltpu.make_async_copy(table_hbm.at[i0], rbuf, gsem).wait()

    lanes_hi = lanes + 16

    def transpose(rows, trans):
        @plsc.parallel_loop(0, _G * 8, step=1, unroll=4)
        def tr_body(g):
            bt = g // 8
            v = g % 8
            bt_v = jnp.full((16,), 0, jnp.int32) + bt
            for t in range(16):
                bi = v * 16 + t
                j = bt * 128 + bi
                bi_v = jnp.full((16,), 0, jnp.int32) + bi
                plsc.store_scatter(
                    trans, [bt_v, lanes, bi_v], rows[j, pl.ds(0, 16)]
                )
                plsc.store_scatter(
                    trans, [bt_v, lanes_hi, bi_v], rows[j, pl.ds(16, 16)]
                )

    def fire_outs(k, trans, osem):
        u = u0 + k
        h = u // _NBTG
        btg = u % _NBTG
        d = None
        for ct in range(_EMBED // 8):
            d = pltpu.async_copy(
                trans.at[:, pl.ds(ct * 8, 8), pl.ds(0, 128)],
                out_hbm.at[h, ct].at[pl.ds(btg * _G, _G)],
                osem,
            )
        return d

    def drain_outs(trans, osem):
        d = pltpu.make_async_copy(
            trans.at[:, pl.ds(0, 8), pl.ds(0, 128)], out_hbm.at[0, 0].at[pl.ds(0, _G)],
            osem,
        )
        for _ in range(_EMBED // 8):
            d.wait()

    start_gather(0, i0, r0, g0)
    start_gather(1, i1, r1, g1)
    wait_gather(r0, g0)
    transpose(r0, t0)
    fire_outs(0, t0, os0)
    start_gather(2, i0, r0, g0)
    wait_gather(r1, g1)
    transpose(r1, t1)
    fire_outs(1, t1, os1)
    start_gather(3, i1, r1, g1)

    def pair_body(j, carry):
        drain_outs(t0, os0)
        wait_gather(r0, g0)
        transpose(r0, t0)
        fire_outs(2 * j, t0, os0)
        start_gather(2 * j + 2, i0, r0, g0)
        drain_outs(t1, os1)
        wait_gather(r1, g1)
        transpose(r1, t1)
        fire_outs(2 * j + 1, t1, os1)
        start_gather(2 * j + 3, i1, r1, g1)
        return carry

    lax.fori_loop(1, _U_PER_W // 2 - 1, pair_body, 0)

    drain_outs(t0, os0)
    wait_gather(r0, g0)
    transpose(r0, t0)
    fire_outs(_U_PER_W - 2, t0, os0)
    drain_outs(t1, os1)
    wait_gather(r1, g1)
    transpose(r1, t1)
    fire_outs(_U_PER_W - 1, t1, os1)
    drain_outs(t0, os0)
    drain_outs(t1, os1)


def kernel(indices, embedding_matrix):
    idx_t = indices.T
    out5 = _gather_kernel(idx_t, embedding_matrix)
    return out5.transpose(2, 4, 0, 1, 3).reshape(_BATCH, _HIST, _EMBED)


# speedup vs baseline: 1.0560x; 1.0560x over previous
import functools

import jax
import jax.numpy as jnp
from jax import lax
from jax.experimental import pallas as pl
from jax.experimental.pallas import tpu as pltpu
from jax.experimental.pallas import tpu_sc as plsc

_VOCAB = 1000000
_EMBED = 32
_BATCH = 16384
_HIST = 50

_NC = 2
_NS = 16
_NW = _NC * _NS

_G = 4
_CHUNK = _G * 128
_NBTG = _BATCH // _CHUNK
_NUNIT = _HIST * _NBTG
_U_PER_W = _NUNIT // _NW

_mesh = plsc.VectorSubcoreMesh(core_axis_name="c", subcore_axis_name="s")


@functools.partial(
    pl.kernel,
    mesh=_mesh,
    out_type=jax.ShapeDtypeStruct(
        (_HIST, _EMBED // 8, _BATCH // 128, 8, 128), jnp.float32
    ),
    scratch_types=[
        pltpu.VMEM((_CHUNK,), jnp.int32),
        pltpu.VMEM((_CHUNK,), jnp.int32),
        pltpu.VMEM((_CHUNK, _EMBED), jnp.float32),
        pltpu.VMEM((_CHUNK, _EMBED), jnp.float32),
        pltpu.VMEM((_G, _EMBED, 129), jnp.float32),
        pltpu.VMEM((_G, _EMBED, 129), jnp.float32),
        pltpu.SemaphoreType.DMA,
        pltpu.SemaphoreType.DMA,
        pltpu.SemaphoreType.DMA,
        pltpu.SemaphoreType.DMA,
    ],
    compiler_params=pltpu.CompilerParams(
        use_tc_tiling_on_sc=False, needs_layout_passes=False
    ),
)
def _gather_kernel(idx_hbm, table_hbm, out_hbm, i0, i1, r0, r1, t0, t1,
                   g0, g1, os0, os1):
    wid = lax.axis_index("s") * _NC + lax.axis_index("c")
    u0 = wid * _U_PER_W
    lanes = lax.iota(jnp.int32, 16)

    def start_gather(k, ibuf, rbuf, gsem):
        u = u0 + k
        h = u // _NBTG
        btg = u % _NBTG
        pltpu.sync_copy(idx_hbm.at[h].at[pl.ds(btg * _CHUNK, _CHUNK)], ibuf)
        return pltpu.async_copy(table_hbm.at[ibuf], rbuf, gsem)

    def wait_gather(rbuf, gsem):
        pltpu.make_async_copy(table_hbm.at[i0], rbuf, gsem).wait()

    lanes_hi = lanes + 16

    def transpose(rows, trans):
        @plsc.parallel_loop(0, _G * 8, step=1, unroll=2)
        def tr_body(g):
            bt = g // 8
            v = g % 8
            bt_v = jnp.full((16,), 0, jnp.int32) + bt
            for t in range(16):
                bi = v * 16 + t
                j = bt * 128 + bi
                bi_v = jnp.full((16,), 0, jnp.int32) + bi
                plsc.store_scatter(
                    trans, [bt_v, lanes, bi_v], rows[j, pl.ds(0, 16)]
                )
                plsc.store_scatter(
                    trans, [bt_v, lanes_hi, bi_v], rows[j, pl.ds(16, 16)]
                )

    def fire_outs(k, trans, osem):
        u = u0 + k
        h = u // _NBTG
        btg = u % _NBTG
        d = None
        for ct in range(_EMBED // 8):
            d = pltpu.async_copy(
                trans.at[:, pl.ds(ct * 8, 8), pl.ds(0, 128)],
                out_hbm.at[h, ct].at[pl.ds(btg * _G, _G)],
                osem,
            )
        return d

    def drain_outs(trans, osem):
        d = pltpu.make_async_copy(
            trans.at[:, pl.ds(0, 8), pl.ds(0, 128)], out_hbm.at[0, 0].at[pl.ds(0, _G)],
            osem,
        )
        for _ in range(_EMBED // 8):
            d.wait()

    start_gather(0, i0, r0, g0)
    start_gather(1, i1, r1, g1)
    wait_gather(r0, g0)
    transpose(r0, t0)
    fire_outs(0, t0, os0)
    start_gather(2, i0, r0, g0)
    wait_gather(r1, g1)
    transpose(r1, t1)
    fire_outs(1, t1, os1)
    start_gather(3, i1, r1, g1)

    def pair_body(j, carry):
        drain_outs(t0, os0)
        wait_gather(r0, g0)
        transpose(r0, t0)
        fire_outs(2 * j, t0, os0)
        start_gather(2 * j + 2, i0, r0, g0)
        drain_outs(t1, os1)
        wait_gather(r1, g1)
        transpose(r1, t1)
        fire_outs(2 * j + 1, t1, os1)
        start_gather(2 * j + 3, i1, r1, g1)
        return carry

    lax.fori_loop(1, _U_PER_W // 2 - 1, pair_body, 0)

    drain_outs(t0, os0)
    wait_gather(r0, g0)
    transpose(r0, t0)
    fire_outs(_U_PER_W - 2, t0, os0)
    drain_outs(t1, os1)
    wait_gather(r1, g1)
    transpose(r1, t1)
    fire_outs(_U_PER_W - 1, t1, os1)
    drain_outs(t0, os0)
    drain_outs(t1, os1)


def kernel(indices, embedding_matrix):
    idx_t = indices.T
    out5 = _gather_kernel(idx_t, embedding_matrix)
    return out5.transpose(2, 4, 0, 1, 3).reshape(_BATCH, _HIST, _EMBED)
